# Initial kernel scaffold; baseline (speedup 1.0000x reference)
import jax
import jax.numpy as jnp
from jax.experimental import pallas as pl


def kernel(r, sh_0, sh_1, sh_2, sh_3, centers, neighbors, initial_center_embedding, W_0, W_1, W_2, W_3):
    raise NotImplementedError("write your pallas kernel here")


# speedup vs baseline: 45.8013x; 45.8013x over previous
import functools

import jax
import jax.numpy as jnp
import numpy as np
from jax import lax
from jax.experimental import pallas as pl
from jax.experimental.pallas import tpu as pltpu
from jax.experimental.pallas import tpu_sc as plsc

_N = 10000
_E = 320000
_L_MAX = 3
_KL = [128, 96, 64, 32]
_NG = 32
_RCUT = 5.0

_C = 512
_NCHUNK = _E // _C
_PACK = 960
_ROFF = [0, 128, 224, 288]
_SHOFF = [0, 1, 4, 9]


_NC, _NS = 2, 16
_NW = _NC * _NS
_EPW = _E // _NW
_GB = 80
_GIT = _EPW // _GB


def _sc_gather_body(table_hbm, idx_hbm, out_hbm, idx_v, rows_v, sem):
    wid = lax.axis_index("s") * _NC + lax.axis_index("c")
    base = wid * _EPW
    pltpu.sync_copy(idx_hbm.at[pl.ds(base, _EPW)], idx_v)

    def step(t, carry):
        off = t * _GB
        pltpu.async_copy(table_hbm.at[idx_v.at[pl.ds(off, _GB)]], rows_v, sem).wait()
        pltpu.sync_copy(rows_v, out_hbm.at[pl.ds(base + off, _GB)])
        return carry

    lax.fori_loop(0, _GIT, step, 0)


def _sc_gather(table, idx):
    mesh = plsc.VectorSubcoreMesh(core_axis_name="c", subcore_axis_name="s")
    kern = functools.partial(
        pl.kernel,
        mesh=mesh,
        out_type=jax.ShapeDtypeStruct((_E, 128), jnp.float32),
        scratch_types=[
            pltpu.VMEM((_EPW,), jnp.int32),
            pltpu.VMEM((_GB, 128), jnp.float32),
            pltpu.SemaphoreType.DMA,
        ],
    )(_sc_gather_body)
    return kern(table, idx)


def _tc_body(cf_ref, cl_ref, r_ref, sh_ref, emb_ref, cen_ref, w_ref, out_ref):
    i = pl.program_id(0)

    @pl.when(i == 0)
    def _zero():
        out_ref[...] = jnp.zeros_like(out_ref)

    r = r_ref[...]
    mu = lax.broadcasted_iota(jnp.int32, (1, _NG), 1).astype(jnp.float32) * (
        _RCUT / (_NG - 1))
    sigma = _RCUT / _NG
    g = jnp.exp(-0.5 * ((r - mu) / sigma) ** 2)
    t = jnp.clip(r * (1.0 / _RCUT), 0.0, 1.0)
    fcut = 0.5 * (jnp.cos(jnp.float32(np.pi) * t) + 1.0)
    g = g * fcut
    radial = jnp.dot(g, w_ref[...], preferred_element_type=jnp.float32)

    emb = emb_ref[...]
    sh = sh_ref[...]

    segs = []
    for l in range(_L_MAX + 1):
        k = _KL[l]
        a = radial[:, _ROFF[l]:_ROFF[l] + k] * emb[:, :k]
        for m in range(2 * l + 1):
            segs.append(sh[:, _SHOFF[l] + m:_SHOFF[l] + m + 1] * a)
    msg = jnp.concatenate(segs, axis=1)

    firstc = cf_ref[i]
    lastc = cl_ref[i]
    o0 = firstc // 8
    n_oct = lastc // 8 - o0 + 1
    cen = cen_ref[...].reshape(1, _C)
    cen_b = jnp.broadcast_to(cen, (8, _C))
    row_ids = lax.broadcasted_iota(jnp.int32, (8, _C), 0)

    def octet(j, carry):
        o = o0 + j
        sel = jnp.where(cen_b == o * 8 + row_ids, 1.0, 0.0)
        d8 = jnp.dot(sel, msg, preferred_element_type=jnp.float32)
        row = pl.multiple_of(o * 8, 8)
        out_ref[pl.ds(row, 8), :] = out_ref[pl.ds(row, 8), :] + d8
        return carry

    lax.fori_loop(0, n_oct, octet, 0)


def _tc_call(cf, cl, r2, shc, emb_g, cen32, wcat, interpret=False):
    grid_spec = pltpu.PrefetchScalarGridSpec(
        num_scalar_prefetch=2,
        grid=(_NCHUNK,),
        in_specs=[
            pl.BlockSpec((_C, 1), lambda i, cf, cl: (i, 0)),
            pl.BlockSpec((_C, 16), lambda i, cf, cl: (i, 0)),
            pl.BlockSpec((_C, 128), lambda i, cf, cl: (i, 0)),
            pl.BlockSpec((_C,), lambda i, cf, cl: (i,)),
            pl.BlockSpec((_NG, 320), lambda i, cf, cl: (0, 0)),
        ],
        out_specs=pl.BlockSpec((_N, _PACK), lambda i, cf, cl: (0, 0)),
    )
    return pl.pallas_call(
        _tc_body,
        grid_spec=grid_spec,
        out_shape=jax.ShapeDtypeStruct((_N, _PACK), jnp.float32),
        compiler_params=pltpu.CompilerParams(
            dimension_semantics=("arbitrary",),
            vmem_limit_bytes=128 * 1024 * 1024,
        ),
        interpret=interpret,
    )(cf, cl, r2, shc, emb_g, cen32, wcat)


def kernel(r, sh_0, sh_1, sh_2, sh_3, centers, neighbors,
           initial_center_embedding, W_0, W_1, W_2, W_3):
    table = initial_center_embedding.reshape(_N, 128)
    idx = neighbors.astype(jnp.int32)
    emb_g = _sc_gather(table, idx)

    cen32 = centers.astype(jnp.int32)
    cf = cen32[::_C]
    cl = cen32[_C - 1::_C]
    shc = jnp.concatenate([sh_0, sh_1, sh_2, sh_3], axis=1)
    r2 = r.reshape(_E, 1)
    wcat = jnp.concatenate([W_0, W_1, W_2, W_3], axis=1)

    packed = _tc_call(cf, cl, r2, shc, emb_g, cen32, wcat)

    d0 = packed[:, 0:128].reshape(_N, 1, 128)
    d1 = packed[:, 128:416].reshape(_N, 3, 96)
    d2 = packed[:, 416:736].reshape(_N, 5, 64)
    d3 = packed[:, 736:960].reshape(_N, 7, 32)
    return (d0, d1, d2, d3)


# speedup vs baseline: 56.4851x; 1.2333x over previous
import functools

import jax
import jax.numpy as jnp
import numpy as np
from jax import lax
from jax.experimental import pallas as pl
from jax.experimental.pallas import tpu as pltpu
from jax.experimental.pallas import tpu_sc as plsc

_N = 10000
_E = 320000
_L_MAX = 3
_KL = [128, 96, 64, 32]
_NG = 32
_RCUT = 5.0

_C = 512
_NCHUNK = _E // _C
_PACK = 960
_ROFF = [0, 128, 224, 288]
_SHOFF = [0, 1, 4, 9]


_NC, _NS = 2, 16
_NW = _NC * _NS
_EPW = _E // _NW
_GB = 80
_GIT = _EPW // _GB


def _sc_gather_body(table_hbm, idx_hbm, out_hbm, idx_v, rows_v, sem):
    wid = lax.axis_index("s") * _NC + lax.axis_index("c")
    base = wid * _EPW
    pltpu.sync_copy(idx_hbm.at[pl.ds(base, _EPW)], idx_v)

    def step(t, carry):
        off = t * _GB
        pltpu.async_copy(table_hbm.at[idx_v.at[pl.ds(off, _GB)]], rows_v, sem).wait()
        pltpu.sync_copy(rows_v, out_hbm.at[pl.ds(base + off, _GB)])
        return carry

    lax.fori_loop(0, _GIT, step, 0)


def _sc_gather(table, idx):
    mesh = plsc.VectorSubcoreMesh(core_axis_name="c", subcore_axis_name="s")
    kern = functools.partial(
        pl.kernel,
        mesh=mesh,
        out_type=jax.ShapeDtypeStruct((_E, 128), jnp.float32),
        scratch_types=[
            pltpu.VMEM((_EPW,), jnp.int32),
            pltpu.VMEM((_GB, 128), jnp.float32),
            pltpu.SemaphoreType.DMA,
        ],
    )(_sc_gather_body)
    return kern(table, idx)


def _tc_body(cf_ref, cl_ref, r_ref, r1_ref, sh_ref, emb_ref, cen_ref, w_ref,
             out_ref):
    i = pl.program_id(0)

    @pl.when(i == 0)
    def _zero():
        out_ref[...] = jnp.zeros_like(out_ref)

    r = r_ref[...]
    mu = lax.broadcasted_iota(jnp.int32, (1, _NG), 1).astype(jnp.float32) * (
        _RCUT / (_NG - 1))
    sigma = _RCUT / _NG
    g = jnp.exp(-0.5 * ((r - mu) / sigma) ** 2)
    radial = jnp.dot(g, w_ref[...], preferred_element_type=jnp.float32)

    t = jnp.clip(r1_ref[...] * (1.0 / _RCUT), 0.0, 1.0)
    x = jnp.float32(np.pi) * (t - 0.5)
    x2 = x * x
    s = x * (1.0 + x2 * (jnp.float32(-1 / 6) + x2 * (jnp.float32(1 / 120)
        + x2 * (jnp.float32(-1 / 5040) + x2 * jnp.float32(1 / 362880)))))
    fcut = 0.5 * (1.0 - s)

    emb = emb_ref[...]
    sh = sh_ref[...]

    segs = []
    for l in range(_L_MAX + 1):
        k = _KL[l]
        a = radial[:, _ROFF[l]:_ROFF[l] + k] * emb[:, :k]
        for m in range(2 * l + 1):
            segs.append(sh[:, _SHOFF[l] + m:_SHOFF[l] + m + 1] * a)
    msg = jnp.concatenate(segs, axis=1)

    firstc = cf_ref[i]
    lastc = cl_ref[i]
    o0 = firstc // 8
    n_oct = lastc // 8 - o0 + 1
    cen = cen_ref[...].reshape(1, _C)
    cen_b = jnp.broadcast_to(cen, (8, _C))
    row_ids = lax.broadcasted_iota(jnp.int32, (8, _C), 0)
    fcut_b = jnp.broadcast_to(fcut.reshape(1, _C), (8, _C))

    def octet(j, carry):
        o = o0 + j
        sel = jnp.where(cen_b == o * 8 + row_ids, fcut_b, 0.0)
        d8 = jnp.dot(sel, msg, preferred_element_type=jnp.float32)
        row = pl.multiple_of(o * 8, 8)
        out_ref[pl.ds(row, 8), :] = out_ref[pl.ds(row, 8), :] + d8
        return carry

    lax.fori_loop(0, n_oct, octet, 0)


def _tc_call(cf, cl, r2, r1, shc, emb_g, cen32, wcat, interpret=False):
    grid_spec = pltpu.PrefetchScalarGridSpec(
        num_scalar_prefetch=2,
        grid=(_NCHUNK,),
        in_specs=[
            pl.BlockSpec((_C, 1), lambda i, cf, cl: (i, 0)),
            pl.BlockSpec((_C,), lambda i, cf, cl: (i,)),
            pl.BlockSpec((_C, 16), lambda i, cf, cl: (i, 0)),
            pl.BlockSpec((_C, 128), lambda i, cf, cl: (i, 0)),
            pl.BlockSpec((_C,), lambda i, cf, cl: (i,)),
            pl.BlockSpec((_NG, 320), lambda i, cf, cl: (0, 0)),
        ],
        out_specs=pl.BlockSpec((_N, _PACK), lambda i, cf, cl: (0, 0)),
    )
    return pl.pallas_call(
        _tc_body,
        grid_spec=grid_spec,
        out_shape=jax.ShapeDtypeStruct((_N, _PACK), jnp.float32),
        compiler_params=pltpu.CompilerParams(
            dimension_semantics=("arbitrary",),
            vmem_limit_bytes=128 * 1024 * 1024,
        ),
        interpret=interpret,
    )(cf, cl, r2, r1, shc, emb_g, cen32, wcat)


def kernel(r, sh_0, sh_1, sh_2, sh_3, centers, neighbors,
           initial_center_embedding, W_0, W_1, W_2, W_3):
    table = initial_center_embedding.reshape(_N, 128)
    idx = neighbors.astype(jnp.int32)
    emb_g = _sc_gather(table, idx)

    cen32 = centers.astype(jnp.int32)
    cf = cen32[::_C]
    cl = cen32[_C - 1::_C]
    shc = jnp.concatenate([sh_0, sh_1, sh_2, sh_3], axis=1)
    r2 = r.reshape(_E, 1)
    wcat = jnp.concatenate([W_0, W_1, W_2, W_3], axis=1)

    packed = _tc_call(cf, cl, r2, r, shc, emb_g, cen32, wcat)

    d0 = packed[:, 0:128].reshape(_N, 1, 128)
    d1 = packed[:, 128:416].reshape(_N, 3, 96)
    d2 = packed[:, 416:736].reshape(_N, 5, 64)
    d3 = packed[:, 736:960].reshape(_N, 7, 32)
    return (d0, d1, d2, d3)


# speedup vs baseline: 77.9219x; 1.3795x over previous
import functools

import jax
import jax.numpy as jnp
import numpy as np
from jax import lax
from jax.experimental import pallas as pl
from jax.experimental.pallas import tpu as pltpu
from jax.experimental.pallas import tpu_sc as plsc

_N = 10000
_E = 320000
_L_MAX = 3
_KL = [128, 96, 64, 32]
_NG = 32
_RCUT = 5.0

_C = 512
_NCHUNK = _E // _C
_PACK = 960
_ROFF = [0, 128, 224, 288]
_SHOFF = [0, 1, 4, 9]


_NC, _NS = 2, 16
_NW = _NC * _NS
_EPW = _E // _NW
_GB = 80
_GIT = _EPW // _GB


def _sc_gather_body(table_hbm, idx_hbm, out_hbm, idx_v, rows_v, sem):
    wid = lax.axis_index("s") * _NC + lax.axis_index("c")
    base = wid * _EPW
    pltpu.sync_copy(idx_hbm.at[pl.ds(base, _EPW)], idx_v)

    def step(t, carry):
        off = t * _GB
        pltpu.async_copy(table_hbm.at[idx_v.at[pl.ds(off, _GB)]], rows_v, sem).wait()
        pltpu.sync_copy(rows_v, out_hbm.at[pl.ds(base + off, _GB)])
        return carry

    lax.fori_loop(0, _GIT, step, 0)


def _sc_gather(table, idx):
    mesh = plsc.VectorSubcoreMesh(core_axis_name="c", subcore_axis_name="s")
    kern = functools.partial(
        pl.kernel,
        mesh=mesh,
        out_type=jax.ShapeDtypeStruct((_E, 128), jnp.float32),
        scratch_types=[
            pltpu.VMEM((_EPW,), jnp.int32),
            pltpu.VMEM((_GB, 128), jnp.float32),
            pltpu.SemaphoreType.DMA,
        ],
    )(_sc_gather_body)
    return kern(table, idx)


def _tc_body(cf_ref, cl_ref, r_ref, r1_ref, sh_ref, emb_ref, cen_ref, w_ref,
             p_ref, q_ref, out_ref):
    i = pl.program_id(0)

    @pl.when(i == 0)
    def _zero():
        out_ref[...] = jnp.zeros_like(out_ref)

    r = r_ref[...]
    mu = lax.broadcasted_iota(jnp.int32, (1, _NG), 1).astype(jnp.float32) * (
        _RCUT / (_NG - 1))
    sigma = _RCUT / _NG
    g = jnp.exp(-0.5 * ((r - mu) / sigma) ** 2)
    radial_big = jnp.dot(g, w_ref[...], preferred_element_type=jnp.float32)

    t = jnp.clip(r1_ref[...] * (1.0 / _RCUT), 0.0, 1.0)
    x = jnp.float32(np.pi) * (t - 0.5)
    x2 = x * x
    s = x * (1.0 + x2 * (jnp.float32(-1 / 6) + x2 * (jnp.float32(1 / 120)
        + x2 * (jnp.float32(-1 / 5040) + x2 * jnp.float32(1 / 362880)))))
    fcut = 0.5 * (1.0 - s)

    sh_big = jnp.dot(sh_ref[...], p_ref[...],
                     preferred_element_type=jnp.float32)
    emb_big = jnp.dot(emb_ref[...], q_ref[...],
                      preferred_element_type=jnp.float32)
    msg = sh_big * radial_big * emb_big

    firstc = cf_ref[i]
    lastc = cl_ref[i]
    o0 = firstc // 8
    n_oct = lastc // 8 - o0 + 1
    cen = cen_ref[...].reshape(1, _C)
    cen_b = jnp.broadcast_to(cen, (8, _C))
    row_ids = lax.broadcasted_iota(jnp.int32, (8, _C), 0)
    fcut_b = jnp.broadcast_to(fcut.reshape(1, _C), (8, _C))

    def octet(j, carry):
        o = o0 + j
        sel = jnp.where(cen_b == o * 8 + row_ids, fcut_b, 0.0)
        d8 = jnp.dot(sel, msg, preferred_element_type=jnp.float32)
        row = pl.multiple_of(o * 8, 8)
        out_ref[pl.ds(row, 8), :] = out_ref[pl.ds(row, 8), :] + d8
        return carry

    lax.fori_loop(0, n_oct, octet, 0)


def _tc_call(cf, cl, r2, r1, shc, emb_g, cen32, wbig, psel, qsel,
             interpret=False):
    grid_spec = pltpu.PrefetchScalarGridSpec(
        num_scalar_prefetch=2,
        grid=(_NCHUNK,),
        in_specs=[
            pl.BlockSpec((_C, 1), lambda i, cf, cl: (i, 0)),
            pl.BlockSpec((_C,), lambda i, cf, cl: (i,)),
            pl.BlockSpec((_C, 16), lambda i, cf, cl: (i, 0)),
            pl.BlockSpec((_C, 128), lambda i, cf, cl: (i, 0)),
            pl.BlockSpec((_C,), lambda i, cf, cl: (i,)),
            pl.BlockSpec((_NG, _PACK), lambda i, cf, cl: (0, 0)),
            pl.BlockSpec((16, _PACK), lambda i, cf, cl: (0, 0)),
            pl.BlockSpec((128, _PACK), lambda i, cf, cl: (0, 0)),
        ],
        out_specs=pl.BlockSpec((_N, _PACK), lambda i, cf, cl: (0, 0)),
    )
    return pl.pallas_call(
        _tc_body,
        grid_spec=grid_spec,
        out_shape=jax.ShapeDtypeStruct((_N, _PACK), jnp.float32),
        compiler_params=pltpu.CompilerParams(
            dimension_semantics=("arbitrary",),
            vmem_limit_bytes=128 * 1024 * 1024,
        ),
        interpret=interpret,
    )(cf, cl, r2, r1, shc, emb_g, cen32, wbig, psel, qsel)


def kernel(r, sh_0, sh_1, sh_2, sh_3, centers, neighbors,
           initial_center_embedding, W_0, W_1, W_2, W_3):
    table = initial_center_embedding.reshape(_N, 128)
    idx = neighbors.astype(jnp.int32)
    emb_g = _sc_gather(table, idx)

    cen32 = centers.astype(jnp.int32)
    cf = cen32[::_C]
    cl = cen32[_C - 1::_C]
    shc = jnp.concatenate([sh_0, sh_1, sh_2, sh_3], axis=1)
    r2 = r.reshape(_E, 1)
    wbig = jnp.concatenate(
        [jnp.tile(w, (1, 2 * l + 1)) for l, w in enumerate([W_0, W_1, W_2, W_3])],
        axis=1)
    psel_np = np.zeros((16, _PACK), np.float32)
    qsel_np = np.zeros((128, _PACK), np.float32)
    col = 0
    for l in range(_L_MAX + 1):
        k = _KL[l]
        for m in range(2 * l + 1):
            psel_np[_SHOFF[l] + m, col:col + k] = 1.0
            qsel_np[np.arange(k), np.arange(col, col + k)] = 1.0
            col += k
    psel = jnp.asarray(psel_np)
    qsel = jnp.asarray(qsel_np)

    packed = _tc_call(cf, cl, r2, r, shc, emb_g, cen32, wbig, psel, qsel)

    d0 = packed[:, 0:128].reshape(_N, 1, 128)
    d1 = packed[:, 128:416].reshape(_N, 3, 96)
    d2 = packed[:, 416:736].reshape(_N, 5, 64)
    d3 = packed[:, 736:960].reshape(_N, 7, 32)
    return (d0, d1, d2, d3)
